# baseline (speedup 1.0000x reference)
import jax
import jax.numpy as jnp
from jax.experimental import pallas as pl
from jax.experimental.pallas import tpu as pltpu

_BB = 512

_LOG2E = 1.4426950408889634


def _agg_kernel(pref_ref, c_ref, tp_ref, tc_ref, out_ref):
    bb, n, d = pref_ref.shape
    p = pref_ref[...]
    cv = c_ref[...] * _LOG2E
    prod = p * cv[:, None, :]
    ones = jnp.ones((d, d), dtype=jnp.float32)
    dw = jax.lax.dot(prod.reshape(bb * n, d), ones).reshape(bb, n, d)
    tw = _LOG2E / jnp.abs(tp_ref[...] - tc_ref[...])
    w = dw + tw[:, :, None]
    e = jnp.exp2(w - jnp.max(w, axis=1, keepdims=True))
    num = jnp.sum(e * p, axis=1)
    z = jnp.sum(e, axis=1)
    out_ref[...] = num / z


@jax.jit
def kernel(pref, c, t_pref, t_c):
    B, N, D = pref.shape
    grid = (B // _BB,)
    out = pl.pallas_call(
        _agg_kernel,
        grid=grid,
        in_specs=[
            pl.BlockSpec((_BB, N, D), lambda i: (i, 0, 0)),
            pl.BlockSpec((_BB, D), lambda i: (i, 0)),
            pl.BlockSpec((_BB, N), lambda i: (i, 0)),
            pl.BlockSpec((_BB, 1), lambda i: (i, 0)),
        ],
        out_specs=pl.BlockSpec((_BB, D), lambda i: (i, 0)),
        out_shape=jax.ShapeDtypeStruct((B, D), pref.dtype),
        compiler_params=pltpu.CompilerParams(
            dimension_semantics=("arbitrary",),
            vmem_limit_bytes=56 * 1024 * 1024,
        ),
        name="softmax_pool_agg",
    )(pref, c[:, 0, :], t_pref[:, 0, :], t_c)
    return out[:, None, :]
